# baseline (speedup 1.0000x reference)
import functools

import jax
import jax.numpy as jnp
from jax import lax
from jax.experimental import pallas as pl
from jax.experimental.pallas import tpu as pltpu
from jax.experimental.pallas import tpu_sc as plsc

D = 64
NC = 2
NS = 16
NW = NC * NS
CB = 256
NBUF = 4
IDX_PER_STREAM = 128
NSTREAM = 2 * CB // IDX_PER_STREAM
SCALE = float(D) ** 0.5


def _sc_embedding_gather(t32, idx_flat, B):
    b_per_w = B // NW
    nch = b_per_w // CB
    mesh = plsc.VectorSubcoreMesh(core_axis_name="c", subcore_axis_name="s")

    @functools.partial(
        pl.kernel,
        out_type=jax.ShapeDtypeStruct((2 * B, 32), jnp.float32),
        mesh=mesh,
        compiler_params=pltpu.CompilerParams(use_tc_tiling_on_sc=False),
        scratch_types=(
            [pltpu.VMEM((b_per_w,), jnp.int32)]
            + [pltpu.VMEM((2 * CB,), jnp.int32) for _ in range(NBUF)]
            + [pltpu.VMEM((2 * CB, 32), jnp.float32) for _ in range(NBUF)]
            + [pltpu.SemaphoreType.DMA for _ in range(2 * NBUF)]
        ),
    )
    def k(t32_hbm, idx_hbm, out_hbm, idx_v, *bufs):
        h = bufs[:NBUF]
        r = bufs[NBUF:2 * NBUF]
        gsem = bufs[2 * NBUF:3 * NBUF]
        ssem = bufs[3 * NBUF:]
        wid = lax.axis_index("s") * NC + lax.axis_index("c")
        base = wid * b_per_w

        pltpu.sync_copy(idx_hbm.at[pl.ds(base, b_per_w)], idx_v)

        lane = lax.broadcasted_iota(jnp.int32, (16,), 0)
        alt = lane & 1
        perm_lo = jnp.expand_dims(lane >> 1, 1)
        perm_hi = jnp.expand_dims(8 + (lane >> 1), 1)
        dnums = lax.GatherDimensionNumbers(
            offset_dims=(), collapsed_slice_dims=(0,), start_index_map=(0,))

        def interleave(iv, perm):
            rep = lax.gather(iv, perm, dnums, slice_sizes=(1,),
                             mode=lax.GatherScatterMode.PROMISE_IN_BOUNDS)
            return rep + rep + alt

        def fire_gathers(g, b):
            off = g * CB
            hb = h[b]

            def hsetup(t, carry):
                iv = idx_v[pl.ds(off + t * 16, 16)]
                hb[pl.ds(t * 32, 16)] = interleave(iv, perm_lo)
                hb[pl.ds(t * 32 + 16, 16)] = interleave(iv, perm_hi)
                return carry

            lax.fori_loop(0, CB // 16, hsetup, 0)
            for j in range(NSTREAM):
                pltpu.async_copy(
                    t32_hbm.at[hb.at[pl.ds(j * IDX_PER_STREAM,
                                           IDX_PER_STREAM)]],
                    r[b].at[pl.ds(j * IDX_PER_STREAM, IDX_PER_STREAM)],
                    gsem[b],
                )

        def wait_gathers(b):
            pltpu.make_async_copy(
                t32_hbm.at[pl.ds(0, 2 * CB)], r[b], gsem[b]).wait()

        def start_store(g, b):
            pltpu.async_copy(
                r[b], out_hbm.at[pl.ds((base + g * CB) * 2, 2 * CB)],
                ssem[b])

        def wait_store(b):
            pltpu.make_async_copy(
                r[b], out_hbm.at[pl.ds(0, 2 * CB)], ssem[b]).wait()

        for b in range(NBUF):
            fire_gathers(b, b)

        def outer(o, carry):
            for b in range(NBUF):
                g = o * NBUF + b
                wait_gathers(b)
                start_store(g, b)

                @pl.when(g + NBUF < nch)
                def _():
                    wait_store(b)
                    fire_gathers(g + NBUF, b)
            return carry

        lax.fori_loop(0, nch // NBUF, outer, 0)
        for b in range(NBUF):
            wait_store(b)

    return k(t32, idx_flat)


def kernel(inputs, shared_weights):
    bsz, seq = inputs.shape
    B = bsz * seq
    vocab = shared_weights.shape[0]
    idx_flat = inputs.astype(jnp.int32).reshape(B)
    flat = lax.optimization_barrier(shared_weights.reshape(vocab * D))
    t32 = flat.reshape(2 * vocab, D // 2)
    out = _sc_embedding_gather(t32, idx_flat, B)
    scale = jnp.where(inputs == 0, 0.0, SCALE)[..., None]
    return out.reshape(bsz, seq, D) * scale


# speedup vs baseline: 1.4838x; 1.4838x over previous
import functools

import jax
import jax.numpy as jnp
from jax import lax
from jax.experimental import pallas as pl
from jax.experimental.pallas import tpu as pltpu
from jax.experimental.pallas import tpu_sc as plsc

D = 64
NC = 2
NS = 16
NW = NC * NS
CB = 128
NBUF = 4
GPR = 4
IDX_PER_STREAM = 128
NSTREAM = GPR * CB // IDX_PER_STREAM
SCALE = float(D) ** 0.5


def _sc_embedding_gather(t32, idx_flat, B):
    b_per_w = B // NW
    nch = b_per_w // CB
    mesh = plsc.VectorSubcoreMesh(core_axis_name="c", subcore_axis_name="s")

    @functools.partial(
        pl.kernel,
        out_type=jax.ShapeDtypeStruct((GPR * B, 32), jnp.float32),
        mesh=mesh,
        compiler_params=pltpu.CompilerParams(use_tc_tiling_on_sc=False),
        scratch_types=(
            [pltpu.VMEM((b_per_w,), jnp.int32)]
            + [pltpu.VMEM((GPR * CB,), jnp.int32) for _ in range(NBUF)]
            + [pltpu.VMEM((GPR * CB, 32), jnp.float32) for _ in range(NBUF)]
            + [pltpu.SemaphoreType.DMA for _ in range(2 * NBUF)]
        ),
    )
    def k(t32_hbm, idx_hbm, out_hbm, idx_v, *bufs):
        h = bufs[:NBUF]
        r = bufs[NBUF:2 * NBUF]
        gsem = bufs[2 * NBUF:3 * NBUF]
        ssem = bufs[3 * NBUF:]
        wid = lax.axis_index("s") * NC + lax.axis_index("c")
        base = wid * b_per_w

        pltpu.sync_copy(idx_hbm.at[pl.ds(base, b_per_w)], idx_v)

        lane = lax.broadcasted_iota(jnp.int32, (16,), 0)
        sub = lane & 3
        perms = [jnp.expand_dims(4 * j + (lane >> 2), 1) for j in range(4)]
        dnums = lax.GatherDimensionNumbers(
            offset_dims=(), collapsed_slice_dims=(0,), start_index_map=(0,))

        def fire_gathers(g, b):
            off = g * CB
            hb = h[b]

            def hsetup(t, carry):
                iv = idx_v[pl.ds(off + t * 16, 16)]
                g0 = jnp.where(iv == 0, 2, iv * 4)
                for j in range(4):
                    rep = lax.gather(
                        g0, perms[j], dnums, slice_sizes=(1,),
                        mode=lax.GatherScatterMode.PROMISE_IN_BOUNDS)
                    hb[pl.ds(t * 64 + j * 16, 16)] = rep + sub
                return carry

            lax.fori_loop(0, CB // 16, hsetup, 0)
            for j in range(NSTREAM):
                pltpu.async_copy(
                    t32_hbm.at[hb.at[pl.ds(j * IDX_PER_STREAM,
                                           IDX_PER_STREAM)]],
                    r[b].at[pl.ds(j * IDX_PER_STREAM, IDX_PER_STREAM)],
                    gsem[b],
                )

        def wait_gathers(b):
            pltpu.make_async_copy(
                t32_hbm.at[pl.ds(0, GPR * CB)], r[b], gsem[b]).wait()

        def start_store(g, b):
            pltpu.async_copy(
                r[b], out_hbm.at[pl.ds((base + g * CB) * GPR, GPR * CB)],
                ssem[b])

        def wait_store(b):
            pltpu.make_async_copy(
                r[b], out_hbm.at[pl.ds(0, GPR * CB)], ssem[b]).wait()

        for b in range(NBUF):
            fire_gathers(b, b)

        def outer(o, carry):
            for b in range(NBUF):
                g = o * NBUF + b
                wait_gathers(b)
                start_store(g, b)

                @pl.when(g + NBUF < nch)
                def _():
                    wait_store(b)
                    fire_gathers(g + NBUF, b)
            return carry

        lax.fori_loop(0, nch // NBUF, outer, 0)
        for b in range(NBUF):
            wait_store(b)

    return k(t32, idx_flat)


def kernel(inputs, shared_weights):
    bsz, seq = inputs.shape
    B = bsz * seq
    vocab = shared_weights.shape[0]
    idx_flat = inputs.astype(jnp.int32).reshape(B)
    wp = jnp.pad(shared_weights, ((0, 0), (0, 128 - D))) * SCALE
    t32 = wp.reshape(GPR * vocab, 32)
    out = _sc_embedding_gather(t32, idx_flat, B)
    return out.reshape(B, 128)[:, :D].reshape(bsz, seq, D)


# speedup vs baseline: 1.7402x; 1.1728x over previous
import functools

import jax
import jax.numpy as jnp
from jax import lax
from jax.experimental import pallas as pl
from jax.experimental.pallas import tpu as pltpu
from jax.experimental.pallas import tpu_sc as plsc

D = 64
NC = 2
NS = 16
NW = NC * NS
CB = 256
NBUF = 4
IDX_PER_STREAM = 128
NSTREAM = CB // IDX_PER_STREAM
SCALE = float(D) ** 0.5


def _sc_embedding_gather(t2v, idx_flat, B):
    b_per_w = B // NW
    nch = b_per_w // CB
    mesh = plsc.VectorSubcoreMesh(core_axis_name="c", subcore_axis_name="s")

    @functools.partial(
        pl.kernel,
        out_type=jax.ShapeDtypeStruct((B, 128), jnp.float32),
        mesh=mesh,
        compiler_params=pltpu.CompilerParams(use_tc_tiling_on_sc=False),
        scratch_types=(
            [pltpu.VMEM((b_per_w,), jnp.int32)]
            + [pltpu.VMEM((CB,), jnp.int32) for _ in range(NBUF)]
            + [pltpu.VMEM((CB, D), jnp.float32) for _ in range(NBUF)]
            + [pltpu.SemaphoreType.DMA for _ in range(2 * NBUF)]
        ),
    )
    def k(t2v_hbm, idx_hbm, out_hbm, idx_v, *bufs):
        h = bufs[:NBUF]
        r = bufs[NBUF:2 * NBUF]
        gsem = bufs[2 * NBUF:3 * NBUF]
        ssem = bufs[3 * NBUF:]
        wid = lax.axis_index("s") * NC + lax.axis_index("c")
        base = wid * b_per_w

        pltpu.sync_copy(idx_hbm.at[pl.ds(base, b_per_w)], idx_v)

        def fire_gathers(g, b):
            off = g * CB
            hb = h[b]

            def hsetup(t, carry):
                iv = idx_v[pl.ds(off + t * 16, 16)]
                hb[pl.ds(t * 16, 16)] = jnp.where(iv == 0, 1, iv * 2)
                return carry

            lax.fori_loop(0, CB // 16, hsetup, 0)
            for j in range(NSTREAM):
                pltpu.async_copy(
                    t2v_hbm.at[hb.at[pl.ds(j * IDX_PER_STREAM,
                                           IDX_PER_STREAM)]],
                    r[b].at[pl.ds(j * IDX_PER_STREAM, IDX_PER_STREAM)],
                    gsem[b],
                )

        def wait_gathers(b):
            pltpu.make_async_copy(
                t2v_hbm.at[pl.ds(0, CB)], r[b], gsem[b]).wait()

        def start_store(g, b):
            pltpu.async_copy(
                r[b], out_hbm.at[pl.ds(base + g * CB, CB), pl.ds(0, D)],
                ssem[b])

        def wait_store(b):
            pltpu.make_async_copy(
                r[b], out_hbm.at[pl.ds(0, CB), pl.ds(0, D)], ssem[b]).wait()

        for b in range(NBUF):
            fire_gathers(b, b)

        def outer(o, carry):
            for b in range(NBUF):
                g = o * NBUF + b
                wait_gathers(b)
                start_store(g, b)

                @pl.when(g + NBUF < nch)
                def _():
                    wait_store(b)
                    fire_gathers(g + NBUF, b)
            return carry

        lax.fori_loop(0, nch // NBUF, outer, 0)
        for b in range(NBUF):
            wait_store(b)

    return k(t2v, idx_flat)


def kernel(inputs, shared_weights):
    bsz, seq = inputs.shape
    B = bsz * seq
    vocab = shared_weights.shape[0]
    idx_flat = inputs.astype(jnp.int32).reshape(B)
    wp = jnp.pad(shared_weights, ((0, 0), (0, 128 - D))) * SCALE
    t2v = wp.reshape(2 * vocab, D)
    out = _sc_embedding_gather(t2v, idx_flat, B)
    return out[:, :D].reshape(bsz, seq, D)
